# baseline (speedup 1.0000x reference)
import jax
import jax.numpy as jnp
from jax import lax
from jax.experimental import pallas as pl
from jax.experimental.pallas import tpu as pltpu
from jax.experimental.pallas import tpu_sc as plsc

_R, _C = 4, 8192
_N = _R * _C
_NC, _NS, _L = 1, 16, 16
_NW = _NC * _NS
_CHUNK = _N // _NW


def _body(x_hbm, out_hbm, x_v, o_v):
    wid = lax.axis_index("s") * _NC + lax.axis_index("c")
    base = wid * _CHUNK
    pltpu.sync_copy(x_hbm.at[pl.ds(base, _CHUNK)], x_v)

    def step(i, carry):
        o_v[pl.ds(i * _L, _L)] = x_v[pl.ds(i * _L, _L)] & 15
        return carry

    lax.fori_loop(0, _CHUNK // _L, step, 0)
    pltpu.sync_copy(o_v, out_hbm.at[pl.ds(base, _CHUNK)])


def kernel(token_types):
    x = token_types.reshape(_N)
    out = pl.kernel(
        _body,
        out_type=jax.ShapeDtypeStruct((_N,), jnp.int32),
        mesh=plsc.VectorSubcoreMesh(
            core_axis_name="c", subcore_axis_name="s", num_cores=_NC
        ),
        scratch_types=[
            pltpu.VMEM((_CHUNK,), jnp.int32),
            pltpu.VMEM((_CHUNK,), jnp.int32),
        ],
        compiler_params=pltpu.CompilerParams(skip_device_barrier=True),
    )(x)
    return out.reshape(_R, _C)


# speedup vs baseline: 1.0713x; 1.0713x over previous
import jax
import jax.numpy as jnp
from jax import lax
from jax.experimental import pallas as pl
from jax.experimental.pallas import tpu as pltpu
from jax.experimental.pallas import tpu_sc as plsc

_R, _C = 4, 8192
_NS, _L = 16, 16
_CHUNK = _R * _C // _NS
_PER_ROW = _C // _CHUNK


def _body(x_hbm, out_hbm, x_v, o_v):
    wid = lax.axis_index("s")
    row = wid // _PER_ROW
    col = (wid % _PER_ROW) * _CHUNK
    pltpu.sync_copy(x_hbm.at[row, pl.ds(col, _CHUNK)], x_v)

    def step(i, carry):
        o_v[pl.ds(i * _L, _L)] = x_v[pl.ds(i * _L, _L)] & 15
        return carry

    lax.fori_loop(0, _CHUNK // _L, step, 0)
    pltpu.sync_copy(o_v, out_hbm.at[row, pl.ds(col, _CHUNK)])


def kernel(token_types):
    return pl.kernel(
        _body,
        out_type=jax.ShapeDtypeStruct((_R, _C), jnp.int32),
        mesh=plsc.VectorSubcoreMesh(
            core_axis_name="c", subcore_axis_name="s", num_cores=1
        ),
        scratch_types=[
            pltpu.VMEM((_CHUNK,), jnp.int32),
            pltpu.VMEM((_CHUNK,), jnp.int32),
        ],
    )(token_types)


# speedup vs baseline: 1.0895x; 1.0169x over previous
import jax
import jax.numpy as jnp
from jax import lax
from jax.experimental import pallas as pl
from jax.experimental.pallas import tpu as pltpu
from jax.experimental.pallas import tpu_sc as plsc

_R, _C = 4, 8192
_NS, _L = 16, 16
_CHUNK = _R * _C // _NS
_PER_ROW = _C // _CHUNK


_H = _CHUNK // 2


def _body(x_hbm, out_hbm, x0, x1, o0, o1, s0, s1, t0, t1):
    wid = lax.axis_index("s")
    row = wid // _PER_ROW
    col = (wid % _PER_ROW) * _CHUNK

    in0 = pltpu.async_copy(x_hbm.at[row, pl.ds(col, _H)], x0, s0)
    in1 = pltpu.async_copy(x_hbm.at[row, pl.ds(col + _H, _H)], x1, s1)

    def make_step(src, dst):
        def step(i, carry):
            dst[pl.ds(i * _L, _L)] = src[pl.ds(i * _L, _L)] & 15
            return carry

        return step

    in0.wait()
    lax.fori_loop(0, _H // _L, make_step(x0, o0), 0)
    out0 = pltpu.async_copy(o0, out_hbm.at[row, pl.ds(col, _H)], t0)
    in1.wait()
    lax.fori_loop(0, _H // _L, make_step(x1, o1), 0)
    out1 = pltpu.async_copy(o1, out_hbm.at[row, pl.ds(col + _H, _H)], t1)
    out0.wait()
    out1.wait()


def kernel(token_types):
    return pl.kernel(
        _body,
        out_type=jax.ShapeDtypeStruct((_R, _C), jnp.int32),
        mesh=plsc.VectorSubcoreMesh(
            core_axis_name="c", subcore_axis_name="s", num_cores=1
        ),
        scratch_types=[
            pltpu.VMEM((_H,), jnp.int32),
            pltpu.VMEM((_H,), jnp.int32),
            pltpu.VMEM((_H,), jnp.int32),
            pltpu.VMEM((_H,), jnp.int32),
            pltpu.SemaphoreType.DMA,
            pltpu.SemaphoreType.DMA,
            pltpu.SemaphoreType.DMA,
            pltpu.SemaphoreType.DMA,
        ],
    )(token_types)


# speedup vs baseline: 1.1173x; 1.0256x over previous
import jax
import jax.numpy as jnp
from jax import lax
from jax.experimental import pallas as pl
from jax.experimental.pallas import tpu as pltpu
from jax.experimental.pallas import tpu_sc as plsc

_R, _C = 4, 8192
_NS, _L = 16, 16
_CHUNK = _R * _C // _NS
_PER_ROW = _C // _CHUNK


_H = _CHUNK // 2


def _body(x_hbm, out_hbm, x0, x1, o0, o1, s0, s1, t0, t1):
    wid = lax.axis_index("s")
    row = wid // _PER_ROW
    col = (wid % _PER_ROW) * _CHUNK

    in0 = pltpu.async_copy(x_hbm.at[row, pl.ds(col, _H)], x0, s0)
    in1 = pltpu.async_copy(x_hbm.at[row, pl.ds(col + _H, _H)], x1, s1)

    def run_loop(src, dst):
        @plsc.parallel_loop(0, _H, step=_L, unroll=4)
        def body(i):
            dst[pl.ds(i, _L)] = src[pl.ds(i, _L)] & 15

    in0.wait()
    run_loop(x0, o0)
    out0 = pltpu.async_copy(o0, out_hbm.at[row, pl.ds(col, _H)], t0)
    in1.wait()
    run_loop(x1, o1)
    out1 = pltpu.async_copy(o1, out_hbm.at[row, pl.ds(col + _H, _H)], t1)
    out0.wait()
    out1.wait()


def kernel(token_types):
    return pl.kernel(
        _body,
        out_type=jax.ShapeDtypeStruct((_R, _C), jnp.int32),
        mesh=plsc.VectorSubcoreMesh(
            core_axis_name="c", subcore_axis_name="s", num_cores=1
        ),
        scratch_types=[
            pltpu.VMEM((_H,), jnp.int32),
            pltpu.VMEM((_H,), jnp.int32),
            pltpu.VMEM((_H,), jnp.int32),
            pltpu.VMEM((_H,), jnp.int32),
            pltpu.SemaphoreType.DMA,
            pltpu.SemaphoreType.DMA,
            pltpu.SemaphoreType.DMA,
            pltpu.SemaphoreType.DMA,
        ],
    )(token_types)
